# baseline (speedup 1.0000x reference)
import functools

import jax
import jax.numpy as jnp
from jax import lax
from jax.experimental import pallas as pl
from jax.experimental.pallas import tpu as pltpu
from jax.experimental.pallas import tpu_sc as plsc

E = 64
K = 8
H = 768


def _scores_body(x_ref, w_ref, s_ref):
    logits = lax.dot_general(
        x_ref[...], w_ref[...],
        dimension_numbers=(((1,), (1,)), ((), ())),
        preferred_element_type=jnp.float32,
    )
    s_ref[...] = jax.nn.sigmoid(logits)


@functools.partial(jax.jit, static_argnames=("block_t",))
def _scores_tc(x2d, W, block_t=1024):
    n_tok = x2d.shape[0]
    return pl.pallas_call(
        _scores_body,
        grid=(n_tok // block_t,),
        in_specs=[
            pl.BlockSpec((block_t, H), lambda i: (i, 0)),
            pl.BlockSpec((E, H), lambda i: (0, 0)),
        ],
        out_specs=pl.BlockSpec((block_t, E), lambda i: (i, 0)),
        out_shape=jax.ShapeDtypeStruct((n_tok, E), jnp.float32),
        compiler_params=pltpu.CompilerParams(
            dimension_semantics=("parallel",),
        ),
    )(x2d, W)


def _sortkv_desc(k, v):
    return plsc.sort_key_val(k, v, descending=True)


def _merge_desc(ak, av, bk, bv):
    rbk = lax.rev(bk, (0,))
    rbv = lax.rev(bv, (0,))
    cmp = ak >= rbk
    hik = jnp.where(cmp, ak, rbk)
    hiv = jnp.where(cmp, av, rbv)
    return _sortkv_desc(hik, hiv)


def _make_sc_topk(n_tok):
    nw = 32
    tpw = n_tok // nw

    mesh = plsc.VectorSubcoreMesh(core_axis_name="c", subcore_axis_name="s")

    @functools.partial(
        pl.kernel,
        mesh=mesh,
        compiler_params=pltpu.CompilerParams(needs_layout_passes=False),
        out_type=[
            jax.ShapeDtypeStruct((n_tok * K,), jnp.int32),
            jax.ShapeDtypeStruct((n_tok * K,), jnp.float32),
        ],
        scratch_types=[
            pltpu.VMEM((tpw * E,), jnp.float32),
            pltpu.VMEM((64,), jnp.float32),
            pltpu.VMEM((tpw * K,), jnp.int32),
            pltpu.VMEM((tpw * K,), jnp.float32),
        ],
    )
    def sc_topk(scores_hbm, bias_hbm, idx_hbm, wgt_hbm, s_v, b_v, i_v, w_v):
        wid = lax.axis_index("s") * 2 + lax.axis_index("c")
        base = wid * tpw
        pltpu.sync_copy(scores_hbm.at[pl.ds(base * E, tpw * E)], s_v)
        pltpu.sync_copy(bias_hbm, b_v)

        lane = lax.broadcasted_iota(jnp.int32, (16,), 0)
        low8 = lane < 8
        bias_c = [b_v[pl.ds(16 * j, 16)] for j in range(4)]
        ids_c = [lane + 16 * j for j in range(4)]

        @plsc.parallel_loop(0, tpw, step=1, unroll=4)
        def body(t):
            off = t * E
            srt = []
            for j in range(4):
                s = s_v[pl.ds(off + 16 * j, 16)]
                srt.append(_sortkv_desc(s + bias_c[j], ids_c[j]))
            m01 = _merge_desc(*srt[0], *srt[1])
            m23 = _merge_desc(*srt[2], *srt[3])
            tk, tv = _merge_desc(*m01, *m23)
            bg = plsc.load_gather(b_v, [tv])
            sc = jnp.where(low8, tk - bg, 0.0)
            total = jnp.sum(sc, axis=0)
            w = sc / total
            dst = lane + t * K
            plsc.store_scatter(i_v, [dst], tv, mask=low8)
            plsc.store_scatter(w_v, [dst], w, mask=low8)

        pltpu.sync_copy(i_v, idx_hbm.at[pl.ds(base * K, tpw * K)])
        pltpu.sync_copy(w_v, wgt_hbm.at[pl.ds(base * K, tpw * K)])

    return sc_topk


@jax.jit
def _router_hybrid(x2d, W, bias):
    n_tok = x2d.shape[0]
    scores = _scores_tc(x2d, W)
    idx_f, wgt_f = _make_sc_topk(n_tok)(scores.reshape(-1), bias)
    return idx_f.reshape(n_tok, K), wgt_f.reshape(n_tok, K)


def kernel(x, W, expert_bias):
    B, S, _ = x.shape
    x2d = x.reshape(B * S, H)
    idx, wgt = _router_hybrid(x2d, W, expert_bias)
    return (idx.reshape(B, S, K), wgt.reshape(B, S, K))


# speedup vs baseline: 1.0769x; 1.0769x over previous
import functools

import jax
import jax.numpy as jnp
from jax import lax
from jax.experimental import pallas as pl
from jax.experimental.pallas import tpu as pltpu
from jax.experimental.pallas import tpu_sc as plsc

E = 64
K = 8
H = 768


def _scores_body(x_ref, w_ref, s_ref):
    logits = lax.dot_general(
        x_ref[...], w_ref[...],
        dimension_numbers=(((1,), (1,)), ((), ())),
        preferred_element_type=jnp.float32,
    )
    s_ref[...] = jax.nn.sigmoid(logits)


@functools.partial(jax.jit, static_argnames=("block_t",))
def _scores_tc(x2d, W, block_t=4096):
    n_tok = x2d.shape[0]
    return pl.pallas_call(
        _scores_body,
        grid=(n_tok // block_t,),
        in_specs=[
            pl.BlockSpec((block_t, H), lambda i: (i, 0)),
            pl.BlockSpec((E, H), lambda i: (0, 0)),
        ],
        out_specs=pl.BlockSpec((block_t, E), lambda i: (i, 0)),
        out_shape=jax.ShapeDtypeStruct((n_tok, E), jnp.float32),
        compiler_params=pltpu.CompilerParams(
            dimension_semantics=("parallel",),
        ),
    )(x2d, W)


def _sortkv_desc(k, v):
    return plsc.sort_key_val(k, v, descending=True)


def _merge_desc(ak, av, bk, bv):
    rbk = lax.rev(bk, (0,))
    rbv = lax.rev(bv, (0,))
    cmp = ak >= rbk
    hik = jnp.where(cmp, ak, rbk)
    hiv = jnp.where(cmp, av, rbv)
    return _sortkv_desc(hik, hiv)


def _make_sc_topk(n_tok):
    nw = 32
    tpw = n_tok // nw

    mesh = plsc.VectorSubcoreMesh(core_axis_name="c", subcore_axis_name="s")

    @functools.partial(
        pl.kernel,
        mesh=mesh,
        compiler_params=pltpu.CompilerParams(needs_layout_passes=False),
        out_type=[
            jax.ShapeDtypeStruct((n_tok * K,), jnp.int32),
            jax.ShapeDtypeStruct((n_tok * K,), jnp.float32),
        ],
        scratch_types=[
            pltpu.VMEM((tpw * E,), jnp.float32),
            pltpu.VMEM((64,), jnp.float32),
            pltpu.VMEM((tpw * K,), jnp.int32),
            pltpu.VMEM((tpw * K,), jnp.float32),
        ],
    )
    def sc_topk(scores_hbm, bias_hbm, idx_hbm, wgt_hbm, s_v, b_v, i_v, w_v):
        wid = lax.axis_index("s") * 2 + lax.axis_index("c")
        base = wid * tpw
        pltpu.sync_copy(scores_hbm.at[pl.ds(base * E, tpw * E)], s_v)
        pltpu.sync_copy(bias_hbm, b_v)

        lane = lax.broadcasted_iota(jnp.int32, (16,), 0)
        low8 = lane < 8
        bias_c = [b_v[pl.ds(16 * j, 16)] for j in range(4)]
        ids_c = [lane + 16 * j for j in range(4)]

        @plsc.parallel_loop(0, tpw, step=1, unroll=4)
        def body(t):
            off = t * E
            srt = []
            for j in range(4):
                s = s_v[pl.ds(off + 16 * j, 16)]
                srt.append(_sortkv_desc(s + bias_c[j], ids_c[j]))
            m01 = _merge_desc(*srt[0], *srt[1])
            m23 = _merge_desc(*srt[2], *srt[3])
            tk, tv = _merge_desc(*m01, *m23)
            bg = plsc.load_gather(b_v, [tv])
            sc = jnp.where(low8, tk - bg, 0.0)
            total = jnp.sum(sc, axis=0)
            w = sc / total
            dst = lane + t * K
            plsc.store_scatter(i_v, [dst], tv, mask=low8)
            plsc.store_scatter(w_v, [dst], w, mask=low8)

        pltpu.sync_copy(i_v, idx_hbm.at[pl.ds(base * K, tpw * K)])
        pltpu.sync_copy(w_v, wgt_hbm.at[pl.ds(base * K, tpw * K)])

    return sc_topk


@jax.jit
def _router_hybrid(x2d, W, bias):
    n_tok = x2d.shape[0]
    scores = _scores_tc(x2d, W)
    idx_f, wgt_f = _make_sc_topk(n_tok)(scores.reshape(-1), bias)
    return idx_f.reshape(n_tok, K), wgt_f.reshape(n_tok, K)


def kernel(x, W, expert_bias):
    B, S, _ = x.shape
    x2d = x.reshape(B * S, H)
    idx, wgt = _router_hybrid(x2d, W, expert_bias)
    return (idx.reshape(B, S, K), wgt.reshape(B, S, K))


# speedup vs baseline: 2.8571x; 2.6532x over previous
import functools

import jax
import jax.numpy as jnp
from jax import lax
from jax.experimental import pallas as pl
from jax.experimental.pallas import tpu as pltpu

E = 64
K = 8
H = 768


def _router_body(x_ref, w_ref, b_ref, idx_ref, wgt_ref):
    logits = lax.dot_general(
        w_ref[...], x_ref[...],
        dimension_numbers=(((1,), (1,)), ((), ())),
        preferred_element_type=jnp.float32,
    )
    scores = jax.nn.sigmoid(logits)
    sel_f = scores + b_ref[...]

    T = scores.shape[1]
    eid = lax.broadcasted_iota(jnp.int32, (E, T), 0)
    sel = sel_f
    neg_inf = jnp.float32(-jnp.inf)

    picked_scores = []
    for k in range(K):
        m = jnp.max(sel, axis=0, keepdims=True)
        is_max = sel == m
        idx = jnp.min(jnp.where(is_max, eid, E), axis=0, keepdims=True)
        hit = eid == idx
        score_k = jnp.sum(jnp.where(hit, scores, 0.0), axis=0, keepdims=True)
        picked_scores.append(score_k)
        idx_ref[k : k + 1, :] = idx
        sel = jnp.where(hit, neg_inf, sel)

    stacked = jnp.concatenate(picked_scores, axis=0)
    total = jnp.sum(stacked, axis=0, keepdims=True)
    wgt_ref[...] = stacked / total


@functools.partial(jax.jit, static_argnames=("block_t",))
def _router(x2d, W, bias, block_t=4096):
    n_tok = x2d.shape[0]
    grid = (n_tok // block_t,)
    idx_t, wgt_t = pl.pallas_call(
        _router_body,
        grid=grid,
        in_specs=[
            pl.BlockSpec((block_t, H), lambda i: (i, 0)),
            pl.BlockSpec((E, H), lambda i: (0, 0)),
            pl.BlockSpec((E, 1), lambda i: (0, 0)),
        ],
        out_specs=[
            pl.BlockSpec((K, block_t), lambda i: (0, i)),
            pl.BlockSpec((K, block_t), lambda i: (0, i)),
        ],
        out_shape=[
            jax.ShapeDtypeStruct((K, n_tok), jnp.int32),
            jax.ShapeDtypeStruct((K, n_tok), jnp.float32),
        ],
        compiler_params=pltpu.CompilerParams(
            dimension_semantics=("parallel",),
        ),
    )(x2d, W, bias)
    return idx_t, wgt_t


def kernel(x, W, expert_bias):
    B, S, _ = x.shape
    x2d = x.reshape(B * S, H)
    idx_t, wgt_t = _router(x2d, W, expert_bias.reshape(E, 1))
    top_k_indices = idx_t.T.reshape(B, S, K)
    top_k_weights = wgt_t.T.reshape(B, S, K)
    return (top_k_indices, top_k_weights)
